# baseline (speedup 1.0000x reference)
import functools

import jax
import jax.numpy as jnp
from jax import lax
from jax.experimental import pallas as pl
from jax.experimental.pallas import tpu as pltpu
from jax.experimental.pallas import tpu_sc as plsc

_B = 64
_S = 512
_DIM = 768
_NTOK = _B * _S

_NC = 2
_NS = 16
_NW = _NC * _NS
_TOK_PER_W = _NTOK // _NW
_CHUNK = 64
_NCHUNK = _TOK_PER_W // _CHUNK


def _sc_gather(table, ids):
  mesh = plsc.VectorSubcoreMesh(core_axis_name="c", subcore_axis_name="s")

  @functools.partial(
      pl.kernel,
      out_type=jax.ShapeDtypeStruct((_NTOK, _DIM), jnp.float32),
      mesh=mesh,
      scratch_types=[
          pltpu.VMEM((_TOK_PER_W,), jnp.int32),
          pltpu.VMEM((2, _CHUNK, _DIM), jnp.float32),
          pltpu.SemaphoreType.DMA,
          pltpu.SemaphoreType.DMA,
          pltpu.SemaphoreType.DMA,
          pltpu.SemaphoreType.DMA,
      ],
  )
  def k(table_hbm, idx_hbm, out_hbm, idx_v, rows_v, g0, g1, o0, o1):
    wid = lax.axis_index("s") * _NC + lax.axis_index("c")
    base = wid * _TOK_PER_W
    pltpu.sync_copy(idx_hbm.at[pl.ds(base, _TOK_PER_W)], idx_v)

    gsem = [g0, g1]
    osem = [o0, o1]

    def gather(i):
      return pltpu.async_copy(
          table_hbm.at[idx_v.at[pl.ds(i * _CHUNK, _CHUNK)]],
          rows_v.at[i % 2],
          gsem[i % 2],
      )

    pend_g = [None, None]
    pend_o = [None, None]
    pend_g[0] = gather(0)
    for i in range(_NCHUNK):
      b = i % 2
      nb = (i + 1) % 2
      if i + 1 < _NCHUNK:
        if pend_o[nb] is not None:
          pend_o[nb].wait()
        pend_g[nb] = gather(i + 1)
      pend_g[b].wait()
      pend_o[b] = pltpu.async_copy(
          rows_v.at[b],
          out_hbm.at[pl.ds(base + i * _CHUNK, _CHUNK)],
          osem[b],
      )
    pend_o[0].wait()
    pend_o[1].wait()

  return k(table, ids)


def _tc_addln(words, ptt, gamma, beta):
  tb = 256
  nblk = _S // tb

  def body(w_ref, p_ref, g_ref, b_ref, o_ref):
    x = w_ref[...] + p_ref[...]
    m = jnp.mean(x, axis=-1, keepdims=True)
    v = jnp.mean(x * x, axis=-1, keepdims=True) - m * m
    r = 1.0 / jnp.sqrt(v + 1e-12)
    o_ref[...] = (x - m) * (r * g_ref[...]) + b_ref[...]

  return pl.pallas_call(
      body,
      grid=(_B * nblk,),
      in_specs=[
          pl.BlockSpec((tb, _DIM), lambda i: (i, 0)),
          pl.BlockSpec((tb, _DIM), lambda i: (i % nblk, 0)),
          pl.BlockSpec((1, _DIM), lambda i: (0, 0)),
          pl.BlockSpec((1, _DIM), lambda i: (0, 0)),
      ],
      out_specs=pl.BlockSpec((tb, _DIM), lambda i: (i, 0)),
      out_shape=jax.ShapeDtypeStruct((_NTOK, _DIM), jnp.float32),
  )(words, ptt, gamma, beta)


def kernel(input_ids, word_embeddings, position_embeddings,
           token_type_embeddings, ln_gamma, ln_beta):
  ids = input_ids.reshape(-1).astype(jnp.int32)
  ptt = position_embeddings + token_type_embeddings[0][None, :]
  words = _sc_gather(word_embeddings, ids)
  out = _tc_addln(
      words,
      ptt,
      ln_gamma.reshape(1, _DIM),
      ln_beta.reshape(1, _DIM),
  )
  return out.reshape(_B, _S, _DIM)


# speedup vs baseline: 1.2896x; 1.2896x over previous
import functools

import jax
import jax.numpy as jnp
from jax import lax
from jax.experimental import pallas as pl
from jax.experimental.pallas import tpu as pltpu
from jax.experimental.pallas import tpu_sc as plsc

_B = 64
_S = 512
_DIM = 768
_NTOK = _B * _S

_NC = 2
_NS = 16
_NW = _NC * _NS
_TOK_PER_W = _NTOK // _NW
_CHUNK = 64
_NCHUNK = _TOK_PER_W // _CHUNK


def _sc_gather(table, ids):
  mesh = plsc.VectorSubcoreMesh(core_axis_name="c", subcore_axis_name="s")

  @functools.partial(
      pl.kernel,
      out_type=jax.ShapeDtypeStruct((_NTOK, _DIM), jnp.float32),
      mesh=mesh,
      scratch_types=[
          pltpu.VMEM((_TOK_PER_W,), jnp.int32),
          pltpu.VMEM((2, _CHUNK, _DIM), jnp.float32),
          pltpu.SemaphoreType.DMA,
          pltpu.SemaphoreType.DMA,
          pltpu.SemaphoreType.DMA,
          pltpu.SemaphoreType.DMA,
      ],
  )
  def k(table_hbm, idx_hbm, out_hbm, idx_v, rows_v, g0, g1, o0, o1):
    wid = lax.axis_index("s") * _NC + lax.axis_index("c")
    base = wid * _TOK_PER_W
    pltpu.sync_copy(idx_hbm.at[pl.ds(base, _TOK_PER_W)], idx_v)

    gsem = [g0, g1]
    osem = [o0, o1]

    def gather(i):
      return pltpu.async_copy(
          table_hbm.at[idx_v.at[pl.ds(i * _CHUNK, _CHUNK)]],
          rows_v.at[i % 2],
          gsem[i % 2],
      )

    pend_g = [None, None]
    pend_o = [None, None]
    pend_g[0] = gather(0)
    for i in range(_NCHUNK):
      b = i % 2
      nb = (i + 1) % 2
      if i + 1 < _NCHUNK:
        if pend_o[nb] is not None:
          pend_o[nb].wait()
        pend_g[nb] = gather(i + 1)
      pend_g[b].wait()
      pend_o[b] = pltpu.async_copy(
          rows_v.at[b],
          out_hbm.at[pl.ds(base + i * _CHUNK, _CHUNK)],
          osem[b],
      )
    pend_o[0].wait()
    pend_o[1].wait()

  return k(table, ids)


def _tc_addln(words, ptt, gamma, beta):
  tb = 512
  nblk = _S // tb

  def body(w_ref, p_ref, g_ref, b_ref, o_ref):
    x = w_ref[...] + p_ref[...]
    m = jnp.mean(x, axis=-1, keepdims=True)
    v = jnp.mean(x * x, axis=-1, keepdims=True) - m * m
    r = 1.0 / jnp.sqrt(v + 1e-12)
    o_ref[...] = (x - m) * (r * g_ref[...]) + b_ref[...]

  return pl.pallas_call(
      body,
      grid=(_B * nblk,),
      in_specs=[
          pl.BlockSpec((tb, _DIM), lambda i: (i, 0)),
          pl.BlockSpec((tb, _DIM), lambda i: (i % nblk, 0)),
          pl.BlockSpec((1, _DIM), lambda i: (0, 0)),
          pl.BlockSpec((1, _DIM), lambda i: (0, 0)),
      ],
      out_specs=pl.BlockSpec((tb, _DIM), lambda i: (i, 0)),
      out_shape=jax.ShapeDtypeStruct((_NTOK, _DIM), jnp.float32),
  )(words, ptt, gamma, beta)


def kernel(input_ids, word_embeddings, position_embeddings,
           token_type_embeddings, ln_gamma, ln_beta):
  ids = input_ids.reshape(-1).astype(jnp.int32)
  ptt = position_embeddings + token_type_embeddings[0][None, :]
  words = _sc_gather(word_embeddings, ids)
  out = _tc_addln(
      words,
      ptt,
      ln_gamma.reshape(1, _DIM),
      ln_beta.reshape(1, _DIM),
  )
  return out.reshape(_B, _S, _DIM)


# speedup vs baseline: 1.3683x; 1.0610x over previous
import functools

import jax
import jax.numpy as jnp
from jax import lax
from jax.experimental import pallas as pl
from jax.experimental.pallas import tpu as pltpu
from jax.experimental.pallas import tpu_sc as plsc

_B = 64
_S = 512
_DIM = 768
_NTOK = _B * _S

_NC = 2
_NS = 16
_NW = _NC * _NS
_CHUNK = 64
_K = 4
_B_PER_K = _B // _K
_TOK_PER_K = _NTOK // _K


def _sc_gather(table, ids, ntok):
  mesh = plsc.VectorSubcoreMesh(core_axis_name="c", subcore_axis_name="s")
  tok_per_w = ntok // _NW
  nchunk = tok_per_w // _CHUNK

  @functools.partial(
      pl.kernel,
      out_type=jax.ShapeDtypeStruct((ntok, _DIM), jnp.float32),
      mesh=mesh,
      scratch_types=[
          pltpu.VMEM((tok_per_w,), jnp.int32),
          pltpu.VMEM((2, _CHUNK, _DIM), jnp.float32),
          pltpu.SemaphoreType.DMA,
          pltpu.SemaphoreType.DMA,
          pltpu.SemaphoreType.DMA,
          pltpu.SemaphoreType.DMA,
      ],
  )
  def k(table_hbm, idx_hbm, out_hbm, idx_v, rows_v, g0, g1, o0, o1):
    wid = lax.axis_index("s") * _NC + lax.axis_index("c")
    base = wid * tok_per_w
    pltpu.sync_copy(idx_hbm.at[pl.ds(base, tok_per_w)], idx_v)

    gsem = [g0, g1]
    osem = [o0, o1]

    def gather(i):
      return pltpu.async_copy(
          table_hbm.at[idx_v.at[pl.ds(i * _CHUNK, _CHUNK)]],
          rows_v.at[i % 2],
          gsem[i % 2],
      )

    pend_g = [None, None]
    pend_o = [None, None]
    pend_g[0] = gather(0)
    for i in range(nchunk):
      b = i % 2
      nb = (i + 1) % 2
      if i + 1 < nchunk:
        if pend_o[nb] is not None:
          pend_o[nb].wait()
        pend_g[nb] = gather(i + 1)
      pend_g[b].wait()
      pend_o[b] = pltpu.async_copy(
          rows_v.at[b],
          out_hbm.at[pl.ds(base + i * _CHUNK, _CHUNK)],
          osem[b],
      )
    pend_o[0].wait()
    pend_o[1].wait()

  return k(table, ids)


def _tc_addln_chunk(words, ptt, gamma, beta, kth, prev):
  def body(*refs):
    w_ref, p_ref, g_ref, b_ref = refs[:4]
    o_ref = refs[-1]
    x = w_ref[...] + p_ref[...]
    m = jnp.mean(x, axis=-1, keepdims=True)
    v = jnp.mean(x * x, axis=-1, keepdims=True) - m * m
    r = 1.0 / jnp.sqrt(v + 1e-12)
    o_ref[...] = (x - m) * (r * g_ref[...]) + b_ref[...]

  in_specs = [
      pl.BlockSpec((_S, _DIM), lambda i: (i, 0)),
      pl.BlockSpec((_S, _DIM), lambda i: (0, 0)),
      pl.BlockSpec((1, _DIM), lambda i: (0, 0)),
      pl.BlockSpec((1, _DIM), lambda i: (0, 0)),
  ]
  args = [words, ptt, gamma, beta]
  aliases = {}
  if prev is not None:
    in_specs.append(pl.BlockSpec(memory_space=pl.ANY))
    args.append(prev)
    aliases = {4: 0}

  return pl.pallas_call(
      body,
      grid=(_B_PER_K,),
      in_specs=in_specs,
      out_specs=pl.BlockSpec((_S, _DIM), lambda i, kth=kth: (kth * _B_PER_K + i, 0)),
      out_shape=jax.ShapeDtypeStruct((_NTOK, _DIM), jnp.float32),
      input_output_aliases=aliases,
  )(*args)


def kernel(input_ids, word_embeddings, position_embeddings,
           token_type_embeddings, ln_gamma, ln_beta):
  ids = input_ids.reshape(-1).astype(jnp.int32)
  ptt = position_embeddings + token_type_embeddings[0][None, :]
  gamma = ln_gamma.reshape(1, _DIM)
  beta = ln_beta.reshape(1, _DIM)

  words = [
      _sc_gather(word_embeddings, lax.dynamic_slice_in_dim(ids, k * _TOK_PER_K, _TOK_PER_K), _TOK_PER_K)
      for k in range(_K)
  ]
  out = None
  for k in range(_K):
    out = _tc_addln_chunk(words[k], ptt, gamma, beta, k, out)
  return out.reshape(_B, _S, _DIM)
